# baseline (speedup 1.0000x reference)
import functools

import jax
import jax.numpy as jnp
from jax import lax
from jax.experimental import pallas as pl
from jax.experimental.pallas import tpu as pltpu
from jax.experimental.pallas import tpu_sc as plsc

_B = 16384
_NC, _NS, _L = 2, 16, 16
_NW = _NC * _NS
_RPW = _B // _NW
_CH = 128
_NCH = _RPW // _CH
_NBK = 2000
_EDP = 32
_ACTW = 336
_R = 1024


def _sc_gather(aid2d, uid2d, aimp_f, uimp_f, ash_p, ush_p):
    f32 = jnp.float32
    i32 = jnp.int32
    mesh = plsc.VectorSubcoreMesh(core_axis_name="c", subcore_axis_name="s")

    @functools.partial(
        pl.kernel,
        out_type=jax.ShapeDtypeStruct((_B, 64), f32),
        mesh=mesh,
        scratch_types=[
            pltpu.VMEM((8, _CH), i32),
            pltpu.VMEM((16, _CH), i32),
            pltpu.VMEM((16, _CH), i32),
            pltpu.VMEM((16, _CH), f32),
            pltpu.VMEM((4 * _CH, _EDP), f32),
            pltpu.VMEM((_CH, 64), f32),
            pltpu.VMEM_SHARED((_NBK, _EDP), f32),
            pltpu.VMEM_SHARED((_NBK, _EDP), f32),
            pltpu.SemaphoreType.DMA,
        ],
        compiler_params=pltpu.CompilerParams(needs_layout_passes=False),
    )
    def k(aid_h, uid_h, aimp_h, uimp_h, ash_h, ush_h, pack_h,
          idv, wei, hbi, wv, sbuf, packv, ash_s, ush_s, sem):
        sid = lax.axis_index("s")
        wid = sid * _NC + lax.axis_index("c")
        base = wid * _RPW
        crow = wid * _NCH

        @pl.when(sid == 0)
        def _stage():
            pltpu.sync_copy(ash_h, ash_s)
            pltpu.sync_copy(ush_h, ush_s)

        pltpu.sync_copy(aid_h.at[pl.ds(crow, _NCH)], idv.at[pl.ds(0, _NCH)])
        pltpu.sync_copy(uid_h.at[pl.ds(crow, _NCH)],
                        idv.at[pl.ds(_NCH, _NCH)])
        def hbody(r, _):
            for c in range(_CH // _L):
                s = pl.ds(c * _L, _L)
                a = idv[r, s]
                wei[r, s] = a * 2
                wei[_NCH + r, s] = a * 2 + 1
                hbi[r, s] = (a * 97) % _NBK
                hbi[_NCH + r, s] = (a * 131 + 1) % _NBK
                u = idv[_NCH + r, s]
                wei[2 * _NCH + r, s] = u * 2
                wei[3 * _NCH + r, s] = u * 2 + 1
                hbi[2 * _NCH + r, s] = (u * 97) % _NBK
                hbi[3 * _NCH + r, s] = (u * 131 + 1) % _NBK
            return 0

        lax.fori_loop(0, _NCH, hbody, 0)
        for g in range(4):
            imp = aimp_h if g < 2 else uimp_h
            wcp = [pltpu.async_copy(imp.at[wei.at[g * _NCH + r]],
                                    wv.at[g * _NCH + r], sem)
                   for r in range(_NCH)]
            for cp in wcp:
                cp.wait()
        plsc.subcore_barrier()
        for c in range(_NCH):
            scp = [
                pltpu.async_copy(ash_s.at[hbi.at[c]],
                                 sbuf.at[pl.ds(0, _CH)], sem),
                pltpu.async_copy(ash_s.at[hbi.at[_NCH + c]],
                                 sbuf.at[pl.ds(_CH, _CH)], sem),
                pltpu.async_copy(ush_s.at[hbi.at[2 * _NCH + c]],
                                 sbuf.at[pl.ds(2 * _CH, _CH)], sem),
                pltpu.async_copy(ush_s.at[hbi.at[3 * _NCH + c]],
                                 sbuf.at[pl.ds(3 * _CH, _CH)], sem),
            ]
            for cp in scp:
                cp.wait()

            def body(r, _):
                rf = jnp.full((_L,), r, dtype=i32)

                def wrow(g):
                    return plsc.load_gather(
                        wv, [jnp.full((_L,), g * _NCH + c, dtype=i32), rf])

                w0a = wrow(0)
                w1a = wrow(1)
                w0u = wrow(2)
                w1u = wrow(3)
                for cc in range(2):
                    sl = pl.ds(cc * _L, _L)
                    packv[r, sl] = (w0a * sbuf[r, sl]
                                    + w1a * sbuf[_CH + r, sl])
                    dl = pl.ds(_EDP + cc * _L, _L)
                    packv[r, dl] = (w0u * sbuf[2 * _CH + r, sl]
                                    + w1u * sbuf[3 * _CH + r, sl])
                return 0

            lax.fori_loop(0, _CH, body, 0)
            pltpu.sync_copy(packv, pack_h.at[pl.ds(base + c * _CH, _CH)])

    return k(aid2d, uid2d, aimp_f, uimp_f, ash_p, ush_p)


def _tc_body(feed_r, sv_r, uvi_r, sus_r, sd_r, df_r, pack_r,
             wf_r, wau_r, wsv_r, wuvi_r, wsd_r, wd_r, b_r,
             w1_r, b1_r, w2_r, b2_r, out_r):
    f32 = jnp.float32

    def dot(a, b):
        return lax.dot_general(a, b, (((1,), (0,)), ((), ())),
                               preferred_element_type=f32)

    acts = (dot(feed_r[...], wf_r[...]) + dot(pack_r[...], wau_r[...])
            + dot(sv_r[...], wsv_r[...]) + dot(uvi_r[...], wuvi_r[...])
            + dot(sd_r[...], wsd_r[...]) + dot(df_r[...], wd_r[...])
            + b_r[...])
    e = jnp.maximum(acts[:, 0:320], 0.0)
    g0 = jax.nn.sigmoid(acts[:, 320:324])
    vg = jax.nn.sigmoid(acts[:, 324:328])
    ug = jax.nn.sigmoid(acts[:, 328:332])
    dg = jax.nn.sigmoid(acts[:, 332:336])
    su = (1.0 + vg + ug + dg) * sus_r[...]
    es = e[:, 256:320]
    h = b1_r[...]
    for t in range(4):
        xt = g0[:, t:t + 1] * e[:, 64 * t:64 * t + 64] + (1.0 - g0[:, t:t + 1]) * es
        h = h + dot(xt, w1_r[64 * t:64 * t + 64, :])
    h = jnp.maximum(h, 0.0)
    p1 = jax.nn.sigmoid(dot(h, w2_r[...]) + b2_r[...])
    out_r[...] = jnp.concatenate([2.0 - p1 - su, p1 + su], axis=1)


def _tc_dense(feed, sv, uvi, sus, sd, df, pack,
              wf, wau, wsv, wuvi, wsd, wd, bias, w1, b1, w2, b2):
    nblk = _B // _R

    def row_spec(k):
        return pl.BlockSpec((_R, k), lambda i: (i, 0))

    def full_spec(shape):
        return pl.BlockSpec(shape, lambda i: (0, 0))

    in_specs = [
        row_spec(512), row_spec(4), row_spec(3), row_spec(4), row_spec(4),
        row_spec(1), row_spec(64),
        full_spec((512, _ACTW)), full_spec((64, _ACTW)),
        full_spec((4, _ACTW)), full_spec((3, _ACTW)), full_spec((4, _ACTW)),
        full_spec((1, _ACTW)), full_spec((1, _ACTW)),
        full_spec((256, 128)), full_spec((1, 128)),
        full_spec((128, 4)), full_spec((1, 4)),
    ]
    return pl.pallas_call(
        _tc_body,
        grid=(nblk,),
        in_specs=in_specs,
        out_specs=pl.BlockSpec((_R, 8), lambda i: (i, 0)),
        out_shape=jax.ShapeDtypeStruct((_B, 8), jnp.float32),
    )(feed, sv, uvi, sus, sd, df, pack,
      wf, wau, wsv, wuvi, wsd, wd, bias, w1, b1, w2, b2)


def _fold_weights(p):
    lev = p["ple"]["levels"][0]
    ws = [lev["task_experts"][t][0][0] for t in range(4)] + \
         [lev["shared_experts"][0][0]]
    bs = [lev["task_experts"][t][0][1] for t in range(4)] + \
         [lev["shared_experts"][0][1]]
    gd = [lev["task_gates"][t][0][0] - lev["task_gates"][t][0][1]
          for t in range(4)]
    gdb = jnp.stack([lev["task_gates"][t][1][0] - lev["task_gates"][t][1][1]
                     for t in range(4)])
    wcat = jnp.concatenate(ws + [jnp.stack(gd)], axis=0)
    z = lambda *s: jnp.zeros(s, jnp.float32)
    hot_t = p["hot_w"].T
    w_uv = wcat[:, 532:536]
    ug_t = p["ugate_w"].T
    t0 = p["dev_table"][0, 0]
    t1 = p["dev_table"][1, 0]
    w_did = wcat[:, 556]

    wf = jnp.concatenate([wcat[:, 20:532].T, p["vgate_w"].T, z(512, 8)], axis=1)
    wa = jnp.concatenate([wcat[:, 0:20].T, z(20, 12)], axis=1)
    wu = jnp.concatenate([wcat[:, 536:556].T, z(20, 12)], axis=1)
    wau = jnp.concatenate([wa, z(12, _ACTW), wu, z(12, _ACTW)])
    wsv = jnp.concatenate([hot_t @ w_uv[:, 0:1].T, z(4, 4),
                           hot_t @ ug_t[0:1, :], z(4, 4)], axis=1)
    wuvi = jnp.concatenate([w_uv[:, 1:4].T, z(3, 4), ug_t[1:4, :], z(3, 4)],
                           axis=1)
    wsd = jnp.concatenate([z(4, 332), p["dgate_w"].T], axis=1)
    wd = jnp.concatenate([((t1 - t0) * w_did)[None, :], z(1, 12)], axis=1)
    bias = jnp.concatenate([jnp.concatenate(bs) + t0 * w_did[0:320],
                            gdb + t0 * w_did[320:324],
                            p["vgate_b"], p["ugate_b"], p["dgate_b"]])[None, :]

    towers = p["ple"]["towers"]
    w1 = z(256, 128)
    b1 = []
    w2 = z(128, 4)
    b2 = []
    for t in range(4):
        w1t, b1t, w2t, b2t = towers[t]
        w1 = w1.at[64 * t:64 * t + 64, 32 * t:32 * t + 32].set(w1t.T)
        b1.append(b1t)
        w2 = w2.at[32 * t:32 * t + 32, t].set(w2t[1] - w2t[0])
        b2.append(b2t[1] - b2t[0])
    b1 = jnp.concatenate(b1)[None, :]
    b2 = jnp.stack(b2)[None, :]
    return wf, wau, wsv, wuvi, wsd, wd, bias, w1, b1, w2, b2


def kernel(aid, feed_embedding, statistics_v, uv_info, uid, did,
           statistics_u, statistics_d, params):
    p = params
    aid2d = aid.reshape(_B // _CH, _CH)
    uid2d = uid.reshape(_B // _CH, _CH)
    pad = ((0, 0), (0, _EDP - 20))
    pack = _sc_gather(
        aid2d, uid2d,
        p["aid_imp"].reshape(-1), p["uid_imp"].reshape(-1),
        jnp.pad(p["aid_shared"], pad), jnp.pad(p["uid_shared"], pad))
    folded = _fold_weights(p)
    df = did.astype(jnp.float32)[:, None]
    out8 = _tc_dense(feed_embedding, statistics_v, uv_info, statistics_u,
                     statistics_d, df, pack, *folded)
    return jnp.stack([out8[:, 0:4].T, out8[:, 4:8].T], axis=-1)
